# baseline (speedup 1.0000x reference)
import jax
import jax.numpy as jnp
from jax import lax
from jax.experimental import pallas as pl
from jax.experimental.pallas import tpu as pltpu
from jax.experimental.pallas import tpu_sc as plsc

B = 1024
L = 200
D = 128
NR = 4
NE = 301
NW = 32
ROWS_PER_W = B // NW
LP = 208
HW = D // 2
S0, S1, S2 = 64, 64, 72


def _body(ids_hbm, rt_hbm, et_hbm, lt_hbm, pt_hbm, out_hbm,
          rt_v, et_v, lt_v, pos_v, idx0, idx1, buf0, buf1, buf2,
          sst0, sst1, sst2, spf0, spf1, spf2, sem_i0, sem_i1):
    sid = lax.axis_index("s")
    wid = sid * 2 + lax.axis_index("c")
    row0 = wid * ROWS_PER_W

    @pl.when(sid == 0)
    def _():
        pltpu.sync_copy(pt_hbm, pos_v)
    pltpu.sync_copy(rt_hbm, rt_v)
    pltpu.sync_copy(et_hbm, et_v)
    pltpu.sync_copy(lt_hbm, lt_v)
    plsc.subcore_barrier()
    pltpu.async_copy(ids_hbm.at[row0], idx0, sem_i0)
    pltpu.async_copy(ids_hbm.at[row0 + 1], idx1, sem_i1)
    pltpu.async_copy(pos_v.at[pl.ds(0, S0)], buf0, spf0)
    pltpu.async_copy(pos_v.at[pl.ds(S0, S1)], buf1, spf1)

    def group(idx_v, buf, segbase, goff, ntok):
        rv = idx_v[0, pl.ds(segbase + goff, 16)]
        ev = idx_v[1, pl.ds(segbase + goff, 16)]
        lv = idx_v[2, pl.ds(segbase + goff, 16)]
        for t in range(ntok):
            ro = rv[t]
            eo = ev[t]
            lo_ = lv[t]
            arow = goff + t
            rl = [rt_v[pl.ds(ro + c * 16, 16)] for c in range(4)]
            el = [et_v[pl.ds(eo + c * 16, 16)] for c in range(4)]
            ll = [lt_v[pl.ds(lo_ + c * 16, 16)] for c in range(4)]

            def flo(w):
                return lax.bitcast_convert_type(w << 16, jnp.float32)

            def fhi(w):
                return lax.bitcast_convert_type(w, jnp.float32)

            for c in range(4):
                plsc.addupdate(buf.at[arow, pl.ds(c * 32, 16)],
                               (flo(rl[c]) + flo(el[c])) + flo(ll[c]))
                plsc.addupdate(buf.at[arow, pl.ds(c * 32 + 16, 16)],
                               (fhi(rl[c]) + fhi(el[c])) + fhi(ll[c]))

    def do_row(b, idx_v, i):
        dst0 = out_hbm.at[b, pl.ds(0, S0)]
        dst1 = out_hbm.at[b, pl.ds(S0, S1)]
        dst2 = out_hbm.at[b, pl.ds(S0 + S1, S2)]

        @pl.when(i > 0)
        def _():
            pltpu.make_async_copy(buf2, dst2, sst2).wait()
        pltpu.async_copy(pos_v.at[pl.ds(S0 + S1, S2)], buf2, spf2)

        pltpu.make_async_copy(pos_v.at[pl.ds(0, S0)], buf0, spf0).wait()

        @plsc.parallel_loop(0, S0 // 16)
        def g0(g):
            group(idx_v, buf0, 0, g * 16, 16)
        pltpu.async_copy(buf0, dst0, sst0)

        pltpu.make_async_copy(pos_v.at[pl.ds(S0, S1)], buf1, spf1).wait()

        @plsc.parallel_loop(0, S1 // 16)
        def g1(g):
            group(idx_v, buf1, S0, g * 16, 16)
        pltpu.async_copy(buf1, dst1, sst1)

        pltpu.make_async_copy(buf0, dst0, sst0).wait()
        pltpu.async_copy(pos_v.at[pl.ds(0, S0)], buf0, spf0)

        pltpu.make_async_copy(pos_v.at[pl.ds(S0 + S1, S2)], buf2, spf2).wait()

        @plsc.parallel_loop(0, 4)
        def g2(g):
            group(idx_v, buf2, S0 + S1, g * 16, 16)
        group(idx_v, buf2, S0 + S1, 64, 8)
        pltpu.async_copy(buf2, dst2, sst2)

        pltpu.make_async_copy(buf1, dst1, sst1).wait()
        pltpu.async_copy(pos_v.at[pl.ds(S0, S1)], buf1, spf1)

    def pair_body(j, carry):
        b0 = row0 + 2 * j
        pltpu.make_async_copy(ids_hbm.at[b0], idx0, sem_i0).wait()
        do_row(b0, idx0, 2 * j)
        nxt0 = jnp.minimum(b0 + 2, B - 1)
        pltpu.async_copy(ids_hbm.at[nxt0], idx0, sem_i0)
        pltpu.make_async_copy(ids_hbm.at[b0 + 1], idx1, sem_i1).wait()
        do_row(b0 + 1, idx1, 2 * j + 1)
        nxt1 = jnp.minimum(b0 + 3, B - 1)
        pltpu.async_copy(ids_hbm.at[nxt1], idx1, sem_i1)
        return carry

    lax.fori_loop(0, ROWS_PER_W // 2, pair_body, 0)
    last = row0 + ROWS_PER_W - 1
    pltpu.make_async_copy(ids_hbm.at[last], idx0, sem_i0).wait()
    pltpu.make_async_copy(ids_hbm.at[last], idx1, sem_i1).wait()
    pltpu.make_async_copy(pos_v.at[pl.ds(0, S0)], buf0, spf0).wait()
    pltpu.make_async_copy(pos_v.at[pl.ds(S0, S1)], buf1, spf1).wait()
    pltpu.make_async_copy(
        buf2, out_hbm.at[last, pl.ds(S0 + S1, S2)], sst2).wait()


_sc_call = pl.kernel(
    _body,
    out_type=jax.ShapeDtypeStruct((B, L, D), jnp.float32),
    mesh=plsc.VectorSubcoreMesh(core_axis_name="c", subcore_axis_name="s"),
    scratch_types=[
        pltpu.VMEM((NR * HW,), jnp.int32),
        pltpu.VMEM((NE * HW,), jnp.int32),
        pltpu.VMEM((NE * HW,), jnp.int32),
        pltpu.VMEM_SHARED((L, D), jnp.float32),
        pltpu.VMEM((3, LP), jnp.int32),
        pltpu.VMEM((3, LP), jnp.int32),
        pltpu.VMEM((S0, D), jnp.float32),
        pltpu.VMEM((S1, D), jnp.float32),
        pltpu.VMEM((S2, D), jnp.float32),
        pltpu.SemaphoreType.DMA,
        pltpu.SemaphoreType.DMA,
        pltpu.SemaphoreType.DMA,
        pltpu.SemaphoreType.DMA,
        pltpu.SemaphoreType.DMA,
        pltpu.SemaphoreType.DMA,
        pltpu.SemaphoreType.DMA,
        pltpu.SemaphoreType.DMA,
    ],
)


def _pack_bf16(t):
    tb = t.astype(jnp.bfloat16).reshape(-1, 4, 2, 16)
    tb = tb.transpose(0, 1, 3, 2).reshape(-1)
    return lax.bitcast_convert_type(tb.reshape(-1, 2), jnp.int32).reshape(-1)


@jax.jit
def kernel(response_ids, elapsed_ids, lag_ids, response_table, elapsed_table,
           lag_table, position_table):
    ids = jnp.stack([response_ids.astype(jnp.int32),
                     elapsed_ids.astype(jnp.int32),
                     lag_ids.astype(jnp.int32)], axis=1) * HW
    ids = jnp.pad(ids, ((0, 0), (0, 0), (0, LP - L)))
    return _sc_call(ids, _pack_bf16(response_table),
                    _pack_bf16(elapsed_table), _pack_bf16(lag_table),
                    position_table)


# speedup vs baseline: 1.2089x; 1.2089x over previous
import jax
import jax.numpy as jnp
from jax import lax
from jax.experimental import pallas as pl
from jax.experimental.pallas import tpu as pltpu
from jax.experimental.pallas import tpu_sc as plsc

B = 1024
L = 200
D = 128
NR = 4
NE = 301
NW = 32
ROWS_PER_W = B // NW
LP = 208
HA = 96
HB = 104
HW = D // 2


def _body(ids_hbm, prt_hbm, et_hbm, lt_hbm, out_hbm,
          prt_v, et_v, lt_v, idx0, idx1, acc_a, acc_b,
          sem_a, sem_b, sem_i0, sem_i1):
    wid = lax.axis_index("s") * 2 + lax.axis_index("c")
    row0 = wid * ROWS_PER_W
    pltpu.sync_copy(prt_hbm, prt_v)
    pltpu.sync_copy(et_hbm, et_v)
    pltpu.sync_copy(lt_hbm, lt_v)
    pltpu.async_copy(ids_hbm.at[row0], idx0, sem_i0)
    pltpu.async_copy(ids_hbm.at[row0 + 1], idx1, sem_i1)

    def group(idx_v, acc, base, goff, ntok):
        rv = idx_v[0, pl.ds(base + goff, 16)]
        ev = idx_v[1, pl.ds(base + goff, 16)]
        lv = idx_v[2, pl.ds(base + goff, 16)]
        for t in range(ntok):
            ro = rv[t]
            eo = ev[t]
            lo_ = lv[t]
            arow = goff + t
            rl = [prt_v[pl.ds(ro + c * 16, 16)] for c in range(4)]
            el = [et_v[pl.ds(eo + c * 16, 16)] for c in range(4)]
            ll = [lt_v[pl.ds(lo_ + c * 16, 16)] for c in range(4)]

            def flo(w):
                return lax.bitcast_convert_type(w << 16, jnp.float32)

            def fhi(w):
                return lax.bitcast_convert_type(w, jnp.float32)

            for c in range(4):
                acc[arow, pl.ds(c * 32, 16)] = (
                    (flo(rl[c]) + flo(el[c])) + flo(ll[c]))
                acc[arow, pl.ds(c * 32 + 16, 16)] = (
                    (fhi(rl[c]) + fhi(el[c])) + fhi(ll[c]))

    def do_row(b, idx_v, first):
        dst_a = out_hbm.at[b, pl.ds(0, HA)]
        dst_b = out_hbm.at[b, pl.ds(HA, HB)]

        @pl.when(jnp.logical_not(first))
        def _():
            pltpu.make_async_copy(acc_a, dst_a, sem_a).wait()

        @plsc.parallel_loop(0, HA // 16)
        def ga(g):
            group(idx_v, acc_a, 0, g * 16, 16)
        pltpu.async_copy(acc_a, dst_a, sem_a)

        @pl.when(jnp.logical_not(first))
        def _():
            pltpu.make_async_copy(acc_b, dst_b, sem_b).wait()

        @plsc.parallel_loop(0, HA // 16)
        def gb(g):
            group(idx_v, acc_b, HA, g * 16, 16)
        group(idx_v, acc_b, HA, HA, 8)
        pltpu.async_copy(acc_b, dst_b, sem_b)

    def pair_body(j, carry):
        b0 = row0 + 2 * j
        pltpu.make_async_copy(ids_hbm.at[b0], idx0, sem_i0).wait()
        do_row(b0, idx0, j == 0)
        nxt0 = jnp.minimum(b0 + 2, B - 1)
        pltpu.async_copy(ids_hbm.at[nxt0], idx0, sem_i0)
        pltpu.make_async_copy(ids_hbm.at[b0 + 1], idx1, sem_i1).wait()
        do_row(b0 + 1, idx1, False)
        nxt1 = jnp.minimum(b0 + 3, B - 1)
        pltpu.async_copy(ids_hbm.at[nxt1], idx1, sem_i1)
        return carry

    lax.fori_loop(0, ROWS_PER_W // 2, pair_body, 0)
    last = row0 + ROWS_PER_W - 1
    pltpu.make_async_copy(ids_hbm.at[last], idx0, sem_i0).wait()
    pltpu.make_async_copy(ids_hbm.at[last], idx1, sem_i1).wait()
    pltpu.make_async_copy(acc_a, out_hbm.at[last, pl.ds(0, HA)], sem_a).wait()
    pltpu.make_async_copy(acc_b, out_hbm.at[last, pl.ds(HA, HB)], sem_b).wait()


_sc_call = pl.kernel(
    _body,
    out_type=jax.ShapeDtypeStruct((B, L, D), jnp.float32),
    mesh=plsc.VectorSubcoreMesh(core_axis_name="c", subcore_axis_name="s"),
    scratch_types=[
        pltpu.VMEM((L * NR * HW,), jnp.int32),
        pltpu.VMEM((NE * HW,), jnp.int32),
        pltpu.VMEM((NE * HW,), jnp.int32),
        pltpu.VMEM((3, LP), jnp.int32),
        pltpu.VMEM((3, LP), jnp.int32),
        pltpu.VMEM((HA, D), jnp.float32),
        pltpu.VMEM((HB, D), jnp.float32),
        pltpu.SemaphoreType.DMA,
        pltpu.SemaphoreType.DMA,
        pltpu.SemaphoreType.DMA,
        pltpu.SemaphoreType.DMA,
    ],
)


def _pr_body(pos_ref, resp_ref, out_ref):
    out_ref[...] = pos_ref[...][:, None, :] + resp_ref[...][None, :, :]


_pr_call = pl.pallas_call(
    _pr_body,
    out_shape=jax.ShapeDtypeStruct((L, NR, D), jnp.float32),
)


def _pack_bf16(t):
    tb = t.astype(jnp.bfloat16).reshape(-1, 4, 2, 16)
    tb = tb.transpose(0, 1, 3, 2).reshape(-1)
    return lax.bitcast_convert_type(tb.reshape(-1, 2), jnp.int32).reshape(-1)


@jax.jit
def kernel(response_ids, elapsed_ids, lag_ids, response_table, elapsed_table,
           lag_table, position_table):
    prt = _pr_call(position_table, response_table)
    pr_ids = (response_ids.astype(jnp.int32) +
              jnp.arange(L, dtype=jnp.int32)[None, :] * NR) * HW
    ids = jnp.stack([pr_ids,
                     elapsed_ids.astype(jnp.int32) * HW,
                     lag_ids.astype(jnp.int32) * HW], axis=1)
    ids = jnp.pad(ids, ((0, 0), (0, 0), (0, LP - L)))
    return _sc_call(ids, _pack_bf16(prt.reshape(L * NR, D)),
                    _pack_bf16(elapsed_table), _pack_bf16(lag_table))


# speedup vs baseline: 1.2239x; 1.0124x over previous
import jax
import jax.numpy as jnp
from jax import lax
from jax.experimental import pallas as pl
from jax.experimental.pallas import tpu as pltpu
from jax.experimental.pallas import tpu_sc as plsc

B = 1024
L = 200
D = 128
NR = 4
NE = 301
NW = 32
ROWS_PER_W = B // NW
LP = 208
HA = 96
HB = 104
HW = D // 2


def _copy_ids(rid_hbm, eid_hbm, lid_hbm, b, idx, sem):
    pltpu.async_copy(rid_hbm.at[pl.ds(b * L, L)], idx[0].at[pl.ds(0, L)], sem)
    pltpu.async_copy(eid_hbm.at[pl.ds(b * L, L)], idx[1].at[pl.ds(0, L)], sem)
    pltpu.async_copy(lid_hbm.at[pl.ds(b * L, L)], idx[2].at[pl.ds(0, L)], sem)


def _wait_ids(rid_hbm, b, idx, sem):
    for k in range(3):
        pltpu.make_async_copy(rid_hbm.at[pl.ds(b * L, L)],
                              idx[k].at[pl.ds(0, L)], sem).wait()


def _body(rid_hbm, eid_hbm, lid_hbm, prt_hbm, et_hbm, lt_hbm, out_hbm,
          prt_v, et_v, lt_v, i0r, i0e, i0l, i1r, i1e, i1l, acc_a, acc_b,
          sem_a, sem_b, sem_i0, sem_i1):
    idx0 = (i0r, i0e, i0l)
    idx1 = (i1r, i1e, i1l)
    wid = lax.axis_index("s") * 2 + lax.axis_index("c")
    row0 = wid * ROWS_PER_W
    pltpu.sync_copy(prt_hbm, prt_v)
    pltpu.sync_copy(et_hbm, et_v)
    pltpu.sync_copy(lt_hbm, lt_v)
    _copy_ids(rid_hbm, eid_hbm, lid_hbm, row0, idx0, sem_i0)
    _copy_ids(rid_hbm, eid_hbm, lid_hbm, row0 + 1, idx1, sem_i1)

    def group(idx_v, acc, base, goff, ntok):
        tok0 = base + goff
        liota = (lax.broadcasted_iota(jnp.int32, (16,), 0) + tok0) << 8
        rv = ((idx_v[0][pl.ds(tok0, 16)] & 3) << 6) + liota
        ev = jnp.minimum(idx_v[1][pl.ds(tok0, 16)], NE - 1) << 6
        lv = jnp.minimum(idx_v[2][pl.ds(tok0, 16)], NE - 1) << 6
        for t in range(ntok):
            ro = rv[t]
            eo = ev[t]
            lo_ = lv[t]
            arow = goff + t
            rl = [prt_v[pl.ds(ro + c * 16, 16)] for c in range(4)]
            el = [et_v[pl.ds(eo + c * 16, 16)] for c in range(4)]
            ll = [lt_v[pl.ds(lo_ + c * 16, 16)] for c in range(4)]

            def flo(w):
                return lax.bitcast_convert_type(w << 16, jnp.float32)

            def fhi(w):
                return lax.bitcast_convert_type(w, jnp.float32)

            for c in range(4):
                acc[arow, pl.ds(c * 32, 16)] = (
                    (flo(rl[c]) + flo(el[c])) + flo(ll[c]))
                acc[arow, pl.ds(c * 32 + 16, 16)] = (
                    (fhi(rl[c]) + fhi(el[c])) + fhi(ll[c]))

    def do_row(b, idx_v, first):
        dst_a = out_hbm.at[b, pl.ds(0, HA)]
        dst_b = out_hbm.at[b, pl.ds(HA, HB)]

        @pl.when(jnp.logical_not(first))
        def _():
            pltpu.make_async_copy(acc_a, dst_a, sem_a).wait()

        @plsc.parallel_loop(0, HA // 16)
        def ga(g):
            group(idx_v, acc_a, 0, g * 16, 16)
        pltpu.async_copy(acc_a, dst_a, sem_a)

        @pl.when(jnp.logical_not(first))
        def _():
            pltpu.make_async_copy(acc_b, dst_b, sem_b).wait()

        @plsc.parallel_loop(0, HA // 16)
        def gb(g):
            group(idx_v, acc_b, HA, g * 16, 16)
        group(idx_v, acc_b, HA, HA, 8)
        pltpu.async_copy(acc_b, dst_b, sem_b)

    def pair_body(j, carry):
        b0 = row0 + 2 * j
        _wait_ids(rid_hbm, b0, idx0, sem_i0)
        do_row(b0, idx0, j == 0)
        nxt0 = jnp.minimum(b0 + 2, B - 1)
        _copy_ids(rid_hbm, eid_hbm, lid_hbm, nxt0, idx0, sem_i0)
        _wait_ids(rid_hbm, b0 + 1, idx1, sem_i1)
        do_row(b0 + 1, idx1, False)
        nxt1 = jnp.minimum(b0 + 3, B - 1)
        _copy_ids(rid_hbm, eid_hbm, lid_hbm, nxt1, idx1, sem_i1)
        return carry

    lax.fori_loop(0, ROWS_PER_W // 2, pair_body, 0)
    last = row0 + ROWS_PER_W - 1
    _wait_ids(rid_hbm, last, idx0, sem_i0)
    _wait_ids(rid_hbm, last, idx1, sem_i1)
    pltpu.make_async_copy(acc_a, out_hbm.at[last, pl.ds(0, HA)], sem_a).wait()
    pltpu.make_async_copy(acc_b, out_hbm.at[last, pl.ds(HA, HB)], sem_b).wait()


_sc_call = pl.kernel(
    _body,
    out_type=jax.ShapeDtypeStruct((B, L, D), jnp.float32),
    mesh=plsc.VectorSubcoreMesh(core_axis_name="c", subcore_axis_name="s"),
    scratch_types=[
        pltpu.VMEM((L * NR * HW,), jnp.int32),
        pltpu.VMEM((NE * HW,), jnp.int32),
        pltpu.VMEM((NE * HW,), jnp.int32),
        pltpu.VMEM((LP,), jnp.int32),
        pltpu.VMEM((LP,), jnp.int32),
        pltpu.VMEM((LP,), jnp.int32),
        pltpu.VMEM((LP,), jnp.int32),
        pltpu.VMEM((LP,), jnp.int32),
        pltpu.VMEM((LP,), jnp.int32),
        pltpu.VMEM((HA, D), jnp.float32),
        pltpu.VMEM((HB, D), jnp.float32),
        pltpu.SemaphoreType.DMA,
        pltpu.SemaphoreType.DMA,
        pltpu.SemaphoreType.DMA,
        pltpu.SemaphoreType.DMA,
    ],
)


def _pr_body(pos_ref, resp_ref, out_ref):
    out_ref[...] = pos_ref[...][:, None, :] + resp_ref[...][None, :, :]


_pr_call = pl.pallas_call(
    _pr_body,
    out_shape=jax.ShapeDtypeStruct((L, NR, D), jnp.float32),
)


def _pack_bf16(t):
    tb = t.astype(jnp.bfloat16).reshape(-1, 4, 2, 16)
    tb = tb.transpose(0, 1, 3, 2).reshape(-1)
    return lax.bitcast_convert_type(tb.reshape(-1, 2), jnp.int32).reshape(-1)


@jax.jit
def kernel(response_ids, elapsed_ids, lag_ids, response_table, elapsed_table,
           lag_table, position_table):
    prt = _pr_call(position_table, response_table)
    return _sc_call(response_ids.astype(jnp.int32).reshape(-1),
                    elapsed_ids.astype(jnp.int32).reshape(-1),
                    lag_ids.astype(jnp.int32).reshape(-1),
                    _pack_bf16(prt.reshape(L * NR, D)),
                    _pack_bf16(elapsed_table), _pack_bf16(lag_table))
